# baseline (speedup 1.0000x reference)
import jax
import jax.numpy as jnp
from jax.experimental import pallas as pl
from jax.experimental.pallas import tpu as pltpu

B, L, E, FEAT, D_IN, D_H, D_OUT = 16384, 8, 8, 32, 64, 128, 32
FE = FEAT + 32
XW = D_OUT + FE


def _fused_kernel(feat_ref, p_ref, w0_ref, w1_ref, b1_ref, ids_ref, out_ref):
    bb = feat_ref.shape[0]
    p = p_ref[...]
    feats = feat_ref[...]
    ids = ids_ref[...]
    w0 = w0_ref[...]
    w1 = w1_ref[...]
    b1 = b1_ref[...]

    for n in range(L):
        idn = ids[:, n:n + 1]
        x = jnp.concatenate([p, feats[:, n * FE:(n + 1) * FE]], axis=1)
        pre = jnp.dot(x.astype(jnp.bfloat16), w0,
                      preferred_element_type=jnp.float32)
        psel = pre[:, 0:D_H]
        for i in range(1, E):
            psel = jnp.where(idn == i, pre[:, i * D_H:(i + 1) * D_H], psel)
        h = jnp.tanh(psel)
        o8 = jnp.dot(h.astype(jnp.bfloat16), w1,
                     preferred_element_type=jnp.float32)
        o = o8[:, 0:D_OUT]
        bsel = b1[0:1]
        for i in range(1, E):
            o = jnp.where(idn == i, o8[:, i * D_OUT:(i + 1) * D_OUT], o)
            bsel = jnp.where(idn == i, b1[i:i + 1], bsel)
        p = o + bsel
    out_ref[...] = jnp.maximum(p, 0.0)


def kernel(mod_feat_seq, p_in, W0, b0, W1, b1, mod_id_seq):
    w0cat = jnp.transpose(W0, (2, 0, 1)).reshape(D_IN, E * D_H)
    w0full = jnp.concatenate(
        [w0cat[:D_OUT],
         w0cat[D_OUT:],
         b0.reshape(1, E * D_H),
         jnp.zeros((FE - FEAT - 1, E * D_H), b0.dtype)],
        axis=0).astype(jnp.bfloat16)
    w1all = jnp.transpose(W1, (2, 0, 1)).reshape(D_H, E * D_OUT)
    w1all = w1all.astype(jnp.bfloat16)

    pad = jnp.zeros((B, L, FE - FEAT), mod_feat_seq.dtype)
    pad = pad.at[:, :, 0].set(1.0)
    feats = jnp.concatenate([mod_feat_seq, pad], axis=2).reshape(B, L * FE)
    ids = mod_id_seq.astype(jnp.int32)

    BB = 1024
    grid = (B // BB,)
    return pl.pallas_call(
        _fused_kernel,
        grid=grid,
        in_specs=[
            pl.BlockSpec((BB, L * FE), lambda b: (b, 0)),
            pl.BlockSpec((BB, D_OUT), lambda b: (b, 0)),
            pl.BlockSpec((XW, E * D_H), lambda b: (0, 0)),
            pl.BlockSpec((D_H, E * D_OUT), lambda b: (0, 0)),
            pl.BlockSpec((E, D_OUT), lambda b: (0, 0)),
            pl.BlockSpec((BB, L), lambda b: (b, 0)),
        ],
        out_specs=pl.BlockSpec((BB, D_OUT), lambda b: (b, 0)),
        out_shape=jax.ShapeDtypeStruct((B, D_OUT), jnp.float32),
        compiler_params=pltpu.CompilerParams(
            dimension_semantics=("parallel",)),
    )(feats, p_in, w0full, w1all, b1, ids)
